# baseline (speedup 1.0000x reference)
import jax
import jax.numpy as jnp
from jax import lax
from jax.experimental import pallas as pl
from jax.experimental.pallas import tpu as pltpu

_NSEG = 64
_MU = 0.1
_BLK = 2000


def _moments_kernel(label_ref, data_ref, out_ref, acc_x, acc_sq, acc_c):
    i = pl.program_id(0)
    nsteps = pl.num_programs(0)

    @pl.when(i == 0)
    def _init():
        acc_x[...] = jnp.zeros_like(acc_x)
        acc_sq[...] = jnp.zeros_like(acc_sq)
        acc_c[...] = jnp.zeros_like(acc_c)

    lab = label_ref[...]
    data = data_ref[...]
    seg_ids = lax.broadcasted_iota(jnp.int32, (1, _NSEG), 1)
    ohf = (lab == seg_ids).astype(jnp.float32)

    dn = (((0,), (0,)), ((), ()))
    acc_x[...] += lax.dot_general(ohf, data, dn,
                                  preferred_element_type=jnp.float32)
    rowsq = jnp.sum(data * data, axis=1, keepdims=True)
    acc_sq[...] += jnp.sum(ohf * rowsq, axis=0, keepdims=True)
    acc_c[...] += jnp.sum(ohf, axis=0, keepdims=True)

    @pl.when(i == nsteps - 1)
    def _combine():
        ones11 = jnp.ones((1, 1), jnp.float32)
        tdn = (((0,), (0,)), ((), ()))
        c = lax.dot_general(acc_c[...], ones11, tdn,
                            preferred_element_type=jnp.float32)
        sq_col = lax.dot_general(acc_sq[...], ones11, tdn,
                                 preferred_element_type=jnp.float32)
        safe_c = jnp.maximum(c, 1.0)
        sum_x = acc_x[...]
        ssd = (sq_col
               - jnp.sum(sum_x * sum_x, axis=1, keepdims=True) / safe_c)
        loss2 = ssd / safe_c
        css = c * (c * c - 1.0) / 12.0
        var_idx = css / jnp.maximum(c - 1.0, 1.0)
        std_idx = jnp.sqrt(jnp.maximum(var_idx, 0.0))
        min_std = jnp.sqrt(c * (c + 1.0) / 12.0)
        loss1 = (std_idx - min_std) / safe_c
        present = (c > 0.0).astype(jnp.float32)
        out_ref[0, 0] = jnp.sum(present * (_MU * loss1 + loss2))


def kernel(label, data):
    n, d = data.shape
    grid = n // _BLK
    out = pl.pallas_call(
        _moments_kernel,
        grid=(grid,),
        in_specs=[
            pl.BlockSpec((_BLK, 1), lambda i: (i, 0)),
            pl.BlockSpec((_BLK, d), lambda i: (i, 0)),
        ],
        out_specs=pl.BlockSpec(memory_space=pltpu.SMEM),
        out_shape=jax.ShapeDtypeStruct((1, 1), jnp.float32),
        scratch_shapes=[
            pltpu.VMEM((_NSEG, d), jnp.float32),
            pltpu.VMEM((1, _NSEG), jnp.float32),
            pltpu.VMEM((1, _NSEG), jnp.float32),
        ],
        compiler_params=pltpu.CompilerParams(
            dimension_semantics=("arbitrary",),
        ),
    )(label.reshape(n, 1), data)
    return out[0, 0]


# speedup vs baseline: 1.2802x; 1.2802x over previous
import jax
import jax.numpy as jnp
from jax import lax
from jax.experimental import pallas as pl
from jax.experimental.pallas import tpu as pltpu

_NSEG = 64
_MU = 0.1
_BLK = 8000


def _moments_kernel(label_ref, data_ref, out_ref, acc_x, acc_sq, acc_c):
    i = pl.program_id(0)
    nsteps = pl.num_programs(0)

    @pl.when(i == 0)
    def _init():
        acc_x[...] = jnp.zeros_like(acc_x)
        acc_sq[...] = jnp.zeros_like(acc_sq)
        acc_c[...] = jnp.zeros_like(acc_c)

    lab = label_ref[...]
    data = data_ref[...]
    seg_ids = lax.broadcasted_iota(jnp.int32, (1, _NSEG), 1)
    ohf = (lab == seg_ids).astype(jnp.float32)

    dn = (((0,), (0,)), ((), ()))
    acc_x[...] += lax.dot_general(ohf, data, dn,
                                  preferred_element_type=jnp.float32)
    rowsq = jnp.sum(data * data, axis=1, keepdims=True)
    acc_sq[...] += jnp.sum(ohf * rowsq, axis=0, keepdims=True)
    acc_c[...] += jnp.sum(ohf, axis=0, keepdims=True)

    @pl.when(i == nsteps - 1)
    def _combine():
        ones11 = jnp.ones((1, 1), jnp.float32)
        tdn = (((0,), (0,)), ((), ()))
        c = lax.dot_general(acc_c[...], ones11, tdn,
                            preferred_element_type=jnp.float32)
        sq_col = lax.dot_general(acc_sq[...], ones11, tdn,
                                 preferred_element_type=jnp.float32)
        safe_c = jnp.maximum(c, 1.0)
        sum_x = acc_x[...]
        ssd = (sq_col
               - jnp.sum(sum_x * sum_x, axis=1, keepdims=True) / safe_c)
        loss2 = ssd / safe_c
        css = c * (c * c - 1.0) / 12.0
        var_idx = css / jnp.maximum(c - 1.0, 1.0)
        std_idx = jnp.sqrt(jnp.maximum(var_idx, 0.0))
        min_std = jnp.sqrt(c * (c + 1.0) / 12.0)
        loss1 = (std_idx - min_std) / safe_c
        present = (c > 0.0).astype(jnp.float32)
        out_ref[0, 0] = jnp.sum(present * (_MU * loss1 + loss2))


def kernel(label, data):
    n, d = data.shape
    grid = n // _BLK
    out = pl.pallas_call(
        _moments_kernel,
        grid=(grid,),
        in_specs=[
            pl.BlockSpec((_BLK, 1), lambda i: (i, 0)),
            pl.BlockSpec((_BLK, d), lambda i: (i, 0)),
        ],
        out_specs=pl.BlockSpec(memory_space=pltpu.SMEM),
        out_shape=jax.ShapeDtypeStruct((1, 1), jnp.float32),
        scratch_shapes=[
            pltpu.VMEM((_NSEG, d), jnp.float32),
            pltpu.VMEM((1, _NSEG), jnp.float32),
            pltpu.VMEM((1, _NSEG), jnp.float32),
        ],
        compiler_params=pltpu.CompilerParams(
            dimension_semantics=("arbitrary",),
        ),
    )(label.reshape(n, 1), data)
    return out[0, 0]


# speedup vs baseline: 1.3321x; 1.0406x over previous
import jax
import jax.numpy as jnp
from jax import lax
from jax.experimental import pallas as pl
from jax.experimental.pallas import tpu as pltpu

_NSEG = 64
_MU = 0.1
_BLK = 16000


def _moments_kernel(label_ref, data_ref, out_ref, acc_x, acc_sq, acc_c):
    i = pl.program_id(0)
    nsteps = pl.num_programs(0)

    @pl.when(i == 0)
    def _init():
        acc_x[...] = jnp.zeros_like(acc_x)
        acc_sq[...] = jnp.zeros_like(acc_sq)
        acc_c[...] = jnp.zeros_like(acc_c)

    lab = label_ref[...]
    data = data_ref[...]
    seg_ids = lax.broadcasted_iota(jnp.int32, (1, _NSEG), 1)
    ohf = (lab == seg_ids).astype(jnp.float32)

    dn = (((0,), (0,)), ((), ()))
    acc_x[...] += lax.dot_general(ohf, data, dn,
                                  preferred_element_type=jnp.float32)
    rowsq = jnp.sum(data * data, axis=1, keepdims=True)
    acc_sq[...] += jnp.sum(ohf * rowsq, axis=0, keepdims=True)
    acc_c[...] += jnp.sum(ohf, axis=0, keepdims=True)

    @pl.when(i == nsteps - 1)
    def _combine():
        ones11 = jnp.ones((1, 1), jnp.float32)
        tdn = (((0,), (0,)), ((), ()))
        c = lax.dot_general(acc_c[...], ones11, tdn,
                            preferred_element_type=jnp.float32)
        sq_col = lax.dot_general(acc_sq[...], ones11, tdn,
                                 preferred_element_type=jnp.float32)
        safe_c = jnp.maximum(c, 1.0)
        sum_x = acc_x[...]
        ssd = (sq_col
               - jnp.sum(sum_x * sum_x, axis=1, keepdims=True) / safe_c)
        loss2 = ssd / safe_c
        css = c * (c * c - 1.0) / 12.0
        var_idx = css / jnp.maximum(c - 1.0, 1.0)
        std_idx = jnp.sqrt(jnp.maximum(var_idx, 0.0))
        min_std = jnp.sqrt(c * (c + 1.0) / 12.0)
        loss1 = (std_idx - min_std) / safe_c
        present = (c > 0.0).astype(jnp.float32)
        out_ref[0, 0] = jnp.sum(present * (_MU * loss1 + loss2))


def kernel(label, data):
    n, d = data.shape
    grid = n // _BLK
    out = pl.pallas_call(
        _moments_kernel,
        grid=(grid,),
        in_specs=[
            pl.BlockSpec((_BLK, 1), lambda i: (i, 0)),
            pl.BlockSpec((_BLK, d), lambda i: (i, 0)),
        ],
        out_specs=pl.BlockSpec(memory_space=pltpu.SMEM),
        out_shape=jax.ShapeDtypeStruct((1, 1), jnp.float32),
        scratch_shapes=[
            pltpu.VMEM((_NSEG, d), jnp.float32),
            pltpu.VMEM((1, _NSEG), jnp.float32),
            pltpu.VMEM((1, _NSEG), jnp.float32),
        ],
        compiler_params=pltpu.CompilerParams(
            dimension_semantics=("arbitrary",),
        ),
    )(label.reshape(n, 1), data)
    return out[0, 0]


# speedup vs baseline: 4.9869x; 3.7436x over previous
import jax
import jax.numpy as jnp
from jax import lax
from jax.experimental import pallas as pl
from jax.experimental.pallas import tpu as pltpu

_NSEG = 64
_MU = 0.1
_BLK = 16000


def _moments_kernel(label_ref, data_ref, out_ref, acc_x, acc_sq, acc_c):
    i = pl.program_id(0)
    nsteps = pl.num_programs(0)

    @pl.when(i == 0)
    def _init():
        acc_x[...] = jnp.zeros_like(acc_x)
        acc_sq[...] = jnp.zeros_like(acc_sq)
        acc_c[...] = jnp.zeros_like(acc_c)

    lab = label_ref[0]
    data = data_ref[...]
    seg_ids = lax.broadcasted_iota(jnp.int32, (_NSEG, 1), 0)
    ohf = (lab == seg_ids).astype(jnp.float32)
    ohb = ohf.astype(jnp.bfloat16)

    dn = (((1,), (0,)), ((), ()))
    db = data.astype(jnp.bfloat16)
    acc_x[...] += lax.dot_general(ohb, db, dn,
                                  preferred_element_type=jnp.float32)
    acc_sq[...] += lax.dot_general(ohb, (data * data).astype(jnp.bfloat16), dn,
                                   preferred_element_type=jnp.float32)
    ones_col = jnp.ones((lab.shape[1], 1), jnp.bfloat16)
    acc_c[...] += lax.dot_general(ohb, ones_col, dn,
                                  preferred_element_type=jnp.float32)

    @pl.when(i == nsteps - 1)
    def _combine():
        c = acc_c[...]
        safe_c = jnp.maximum(c, 1.0)
        sum_x = acc_x[...]
        ssd = (jnp.sum(acc_sq[...], axis=1, keepdims=True)
               - jnp.sum(sum_x * sum_x, axis=1, keepdims=True) / safe_c)
        loss2 = ssd / safe_c
        css = c * (c * c - 1.0) / 12.0
        var_idx = css / jnp.maximum(c - 1.0, 1.0)
        std_idx = jnp.sqrt(jnp.maximum(var_idx, 0.0))
        min_std = jnp.sqrt(c * (c + 1.0) / 12.0)
        loss1 = (std_idx - min_std) / safe_c
        present = (c > 0.0).astype(jnp.float32)
        out_ref[0, 0] = jnp.sum(present * (_MU * loss1 + loss2))


def kernel(label, data):
    n, d = data.shape
    grid = n // _BLK
    out = pl.pallas_call(
        _moments_kernel,
        grid=(grid,),
        in_specs=[
            pl.BlockSpec((1, 1, _BLK), lambda i: (i, 0, 0)),
            pl.BlockSpec((_BLK, d), lambda i: (i, 0)),
        ],
        out_specs=pl.BlockSpec(memory_space=pltpu.SMEM),
        out_shape=jax.ShapeDtypeStruct((1, 1), jnp.float32),
        scratch_shapes=[
            pltpu.VMEM((_NSEG, d), jnp.float32),
            pltpu.VMEM((_NSEG, d), jnp.float32),
            pltpu.VMEM((_NSEG, 1), jnp.float32),
        ],
        compiler_params=pltpu.CompilerParams(
            dimension_semantics=("arbitrary",),
        ),
    )(label.reshape(grid, 1, _BLK), data)
    return out[0, 0]
